# baseline (speedup 1.0000x reference)
import functools

import jax
import jax.numpy as jnp
from jax import lax
from jax.experimental import pallas as pl
from jax.experimental.pallas import tpu as pltpu
from jax.experimental.pallas import tpu_sc as plsc

N = 10000
E = 160000
F_IN = 256
HID = 256
HEADS = 8
CLASSES = 10
NGRAPH = 16
EPS_BN = 1e-5
DPAD = 128

NC, NS, L = 2, 16, 16
NW = NC * NS
EP = E + N
CE = 5328
EP_PAD = CE * NW
NP = 10240
RPW = NP // NW
RH = RPW // 2
KMAX = 4096
CS = EP_PAD // 32
DST_PAD = N + 16

_mesh = plsc.VectorSubcoreMesh(core_axis_name="c", subcore_axis_name="s",
                               num_cores=NC, num_subcores=NS)


def _stats1_body(x_ref, mean_ref, rstd_ref):
    x = x_ref[...]
    mean = jnp.mean(x, axis=0, keepdims=True)
    var = jnp.mean((x - mean) ** 2, axis=0, keepdims=True)
    mean_ref[...] = mean
    rstd_ref[...] = lax.rsqrt(var + EPS_BN)


def _l1_body(x_ref, mean_ref, rstd_ref, g_ref, b_ref, w_ref, asr_ref, adr_ref,
             h_ref, as_ref, ad_ref):
    xb = (x_ref[...] - mean_ref[...]) * rstd_ref[...] * g_ref[...] + b_ref[...]
    for hd in range(HEADS):
        w = w_ref[:, hd * HID:(hd + 1) * HID]
        hh = jnp.dot(xb, w, preferred_element_type=jnp.float32)
        h_ref[hd] = hh
        as_ref[0, hd] = jnp.sum(hh * asr_ref[hd][None, :], axis=1)
        ad_ref[0, hd] = jnp.sum(hh * adr_ref[hd][None, :], axis=1)


def _stats2_body(h_ref, bias_ref, mean_ref, rstd_ref):
    y = jnp.maximum(h_ref[0] + bias_ref[0], 0.0)
    rmask = lax.broadcasted_iota(jnp.int32, (NP, 1), 0) < N
    y = jnp.where(rmask, y, 0.0)
    mean = jnp.sum(y, axis=0, keepdims=True) / N
    d = jnp.where(rmask, y - mean, 0.0)
    var = jnp.sum(d * d, axis=0, keepdims=True) / N
    mean_ref[0] = mean
    rstd_ref[0] = lax.rsqrt(var + EPS_BN)


def _l2_body(h_ref, bias_ref, mean_ref, rstd_ref, g_ref, b_ref, w_ref,
             asr_ref, adr_ref, h2_ref, as_ref, ad_ref):
    rb = h_ref.shape[1]
    acc = jnp.zeros((rb, DPAD), jnp.float32)
    for hd in range(HEADS):
        y = jnp.maximum(h_ref[hd] + bias_ref[hd][None, :], 0.0)
        yb = (y - mean_ref[hd][None, :]) * rstd_ref[hd][None, :] \
            * g_ref[hd][None, :] + b_ref[hd][None, :]
        acc = acc + jnp.dot(yb, w_ref[hd], preferred_element_type=jnp.float32)
    h2_ref[0] = acc
    as_ref[0, 0] = jnp.sum(acc * asr_ref[...], axis=1)
    ad_ref[0, 0] = jnp.sum(acc * adr_ref[...], axis=1)


def _final_body(o_ref, bias_ref, batch_ref, out_ref):
    act = o_ref[0] + bias_ref[...]
    act = jnp.where(act > 0.0, act, jnp.exp(act) - 1.0)
    gid = lax.broadcasted_iota(jnp.int32, (NP, NGRAPH), 1)
    oh = (batch_ref[...] == gid).astype(jnp.float32)
    sums = lax.dot_general(oh, act, (((0,), (0,)), ((), ())),
                           preferred_element_type=jnp.float32)
    counts = jnp.sum(oh, axis=0)
    pooled = sums / jnp.maximum(counts, 1.0)[:, None]
    cmask = lax.broadcasted_iota(jnp.int32, (NGRAPH, DPAD), 1) < CLASSES
    pm = jnp.where(cmask, pooled, -1e30)
    mx = jnp.max(pm, axis=1, keepdims=True)
    lse = jnp.log(jnp.sum(jnp.exp(pm - mx), axis=1, keepdims=True)) + mx
    out_ref[...] = pm - lse


def _edge_p_body(heads, src_hbm, dst_hbm, asrc_hbm, adst_hbm,
                 p_hbm, dparts_hbm,
                 src_v, dst_v, asrc_v, adst_v, den_v, p_v):
    wid = lax.axis_index("s") * NC + lax.axis_index("c")
    base = wid * CE
    pltpu.sync_copy(src_hbm.at[pl.ds(base, CE)], src_v)
    pltpu.sync_copy(dst_hbm.at[pl.ds(base, CE)], dst_v)
    zero16 = jnp.zeros((L,), jnp.float32)

    def head_body(hd, c):
        pltpu.sync_copy(asrc_hbm.at[pl.ds(hd * N, N)], asrc_v.at[pl.ds(0, N)])
        pltpu.sync_copy(adst_hbm.at[pl.ds(hd * N, N)], adst_v.at[pl.ds(0, N)])

        def ztail(i, c2):
            adst_v[pl.ds(N + i * L, L)] = zero16
            return c2
        lax.fori_loop(0, (NP - N) // L, ztail, 0)

        def zden(i, c2):
            den_v[pl.ds(i * L, L)] = zero16
            return c2
        lax.fori_loop(0, NP // L, zden, 0)

        def ebody(i, c2):
            sv = src_v[pl.ds(i * L, L)]
            dv = dst_v[pl.ds(i * L, L)]
            av = plsc.load_gather(asrc_v, [sv]) + plsc.load_gather(adst_v, [dv])
            z = jnp.where(av >= 0.0, av, av * 0.2)
            p = jnp.exp(z)
            p_v[pl.ds(i * L, L)] = p
            plsc.addupdate_scatter(den_v, [dv], p)
            return c2
        lax.fori_loop(0, CE // L, ebody, 0)
        pltpu.sync_copy(p_v, p_hbm.at[pl.ds(hd * EP_PAD + base, CE)])
        pltpu.sync_copy(den_v,
                        dparts_hbm.at[pl.ds((wid * heads + hd) * NP, NP)])
        return c
    lax.fori_loop(0, heads, head_body, 0)


def _reduce_body(heads, dparts_hbm, dred_hbm, parts_v, red_v):
    wid = lax.axis_index("s") * NC + lax.axis_index("c")
    lo = wid * RPW

    def head_body(hd, c):
        def ld(k, c2):
            pltpu.sync_copy(
                dparts_hbm.at[pl.ds((k * heads + hd) * NP + lo, RPW)],
                parts_v.at[pl.ds(k * RPW, RPW)])
            return c2
        lax.fori_loop(0, NW, ld, 0)

        def rbody(j, c2):
            acc = jnp.zeros((L,), jnp.float32)
            for k in range(NW):
                acc = acc + parts_v[pl.ds(k * RPW + j * L, L)]
            red_v[pl.ds(j * L, L)] = acc
            return c2
        lax.fori_loop(0, RPW // L, rbody, 0)
        pltpu.sync_copy(red_v, dred_hbm.at[pl.ds(hd * NP + lo, RPW)])
        return c
    lax.fori_loop(0, heads, head_body, 0)


def _norm_body(heads, dst_hbm, p_hbm, dred_hbm, alpha_hbm,
               dst_v, p_v, den_v, al_v):
    wid = lax.axis_index("s") * NC + lax.axis_index("c")
    base = wid * CE
    pltpu.sync_copy(dst_hbm.at[pl.ds(base, CE)], dst_v)

    def head_body(hd, c):
        pltpu.sync_copy(dred_hbm.at[pl.ds(hd * NP, NP)], den_v)
        pltpu.sync_copy(p_hbm.at[pl.ds(hd * EP_PAD + base, CE)], p_v)

        def ebody(i, c2):
            dv = dst_v[pl.ds(i * L, L)]
            p = p_v[pl.ds(i * L, L)]
            dsum = plsc.load_gather(den_v, [dv])
            al_v[pl.ds(i * L, L)] = p / (dsum + 1e-16)
            return c2
        lax.fori_loop(0, CE // L, ebody, 0)
        pltpu.sync_copy(al_v, alpha_hbm.at[pl.ds(hd * EP_PAD + base, CE)])
        return c
    lax.fori_loop(0, heads, head_body, 0)


def _agg_body(heads, d, gb, src_hbm, dst_hbm, alpha_hbm, h_hbm, out_hbm,
              ssrc_v, sdst_v, ksrc_v, kloc_v, keid_v, abuf_v, gbuf_v, acc_v,
              sem0, sem1):
    wid = lax.axis_index("s") * NC + lax.axis_index("c")
    lo = wid * RPW
    lane = lax.iota(jnp.int32, L)
    zero16i = jnp.zeros((L,), jnp.int32)
    zero16f = jnp.zeros((L,), jnp.float32)

    def zk(i, c):
        ksrc_v[pl.ds(i * L, L)] = zero16i
        kloc_v[pl.ds(i * L, L)] = zero16i
        keid_v[pl.ds(i * L, L)] = zero16i
        return c
    lax.fori_loop(0, 2 * KMAX // L, zk, 0)

    def scan_chunk(s, cnts):
        pltpu.sync_copy(src_hbm.at[pl.ds(s * CS, CS)], ssrc_v)
        pltpu.sync_copy(dst_hbm.at[pl.ds(s * CS, CS)], sdst_v)

        def sbody(j, cnts2):
            c0, c1 = cnts2
            sv = ssrc_v[pl.ds(j * L, L)]
            dv = sdst_v[pl.ds(j * L, L)]
            eid = lax.iota(jnp.int32, L) + (s * CS + j * L)
            rel = dv - lo
            m0 = (rel >= 0) & (rel < RH)
            m1 = (rel >= RH) & (rel < RPW)
            plsc.store_compressed(ksrc_v.at[pl.ds(c0, L)], sv, mask=m0)
            plsc.store_compressed(kloc_v.at[pl.ds(c0, L)], rel, mask=m0)
            plsc.store_compressed(keid_v.at[pl.ds(c0, L)], eid, mask=m0)
            plsc.store_compressed(ksrc_v.at[pl.ds(KMAX + c1, L)], sv, mask=m1)
            plsc.store_compressed(kloc_v.at[pl.ds(KMAX + c1, L)], rel - RH,
                                  mask=m1)
            plsc.store_compressed(keid_v.at[pl.ds(KMAX + c1, L)], eid, mask=m1)
            c0 = jnp.minimum(c0 + jnp.sum(m0.astype(jnp.int32)), KMAX - L)
            c1 = jnp.minimum(c1 + jnp.sum(m1.astype(jnp.int32)), KMAX - L)
            return (c0, c1)
        return lax.fori_loop(0, CS // L, sbody, cnts)

    cnt0, cnt1 = lax.fori_loop(0, EP_PAD // CS, scan_chunk,
                               (jnp.int32(0), jnp.int32(0)))

    def run_half(hd, hf, cnt):
        def zacc(i, c):
            acc_v[pl.ds(i * L, L)] = zero16f
            return c
        lax.fori_loop(0, RH * d // L, zacc, 0)

        atab = alpha_hbm.at[pl.ds(hd * EP_PAD, EP_PAD)]
        pltpu.async_copy(atab.at[keid_v.at[pl.ds(hf * KMAX, KMAX)]],
                         abuf_v, sem0).wait()

        htab = h_hbm.at[hd]
        nch = (cnt + gb - 1) // gb

        @pl.when(nch > 0)
        def _():
            idx = ksrc_v.at[pl.ds(hf * KMAX, gb)]
            pltpu.async_copy(htab.at[idx], gbuf_v.at[pl.ds(0, gb)], sem0)

        def chunk(jc, c):
            even = (jc % 2) == 0
            nxt = jc + 1

            @pl.when((nxt < nch) & even)
            def _():
                idx = ksrc_v.at[pl.ds(hf * KMAX + nxt * gb, gb)]
                pltpu.async_copy(htab.at[idx], gbuf_v.at[pl.ds(gb, gb)], sem1)

            @pl.when((nxt < nch) & (~even))
            def _():
                idx = ksrc_v.at[pl.ds(hf * KMAX + nxt * gb, gb)]
                pltpu.async_copy(htab.at[idx], gbuf_v.at[pl.ds(0, gb)], sem0)

            @pl.when(even)
            def _():
                pltpu.make_async_copy(htab.at[pl.ds(0, gb)],
                                      gbuf_v.at[pl.ds(0, gb)], sem0).wait()

            @pl.when(~even)
            def _():
                pltpu.make_async_copy(htab.at[pl.ds(0, gb)],
                                      gbuf_v.at[pl.ds(gb, gb)], sem1).wait()

            rowbase = (jc % 2) * gb

            @plsc.parallel_loop(0, gb // L, unroll=2)
            def group(q):
                ebase = jc * gb + q * L
                lv = kloc_v[pl.ds(hf * KMAX + ebase, L)]
                av = abuf_v[pl.ds(ebase, L)]
                av = jnp.where(ebase + lane < cnt, av, 0.0)
                offv = lv * d
                for r in range(L):
                    msk = lane == r
                    off = jnp.sum(jnp.where(msk, offv, 0))
                    ab = jnp.full((L,), jnp.sum(jnp.where(msk, av, 0.0)))
                    row = rowbase + q * L + r
                    for g in range(d // L):
                        x = gbuf_v[row, pl.ds(g * L, L)].reshape((L,)) * ab
                        plsc.addupdate(acc_v.at[pl.ds(off + g * L, L)], x)
            return c
        lax.fori_loop(0, nch, chunk, 0)
        pltpu.sync_copy(
            acc_v,
            out_hbm.at[pl.ds((hd * NP + lo + hf * RH) * d, RH * d)])

    def head_body(hd, c):
        run_half(hd, 0, cnt0)
        run_half(hd, 1, cnt1)
        return c
    lax.fori_loop(0, heads, head_body, 0)


def _sc_edge_p(heads):
    return functools.partial(
        pl.kernel, functools.partial(_edge_p_body, heads),
        mesh=_mesh,
        compiler_params=pltpu.CompilerParams(needs_layout_passes=False),
        out_type=[jax.ShapeDtypeStruct((heads * EP_PAD,), jnp.float32),
                  jax.ShapeDtypeStruct((NW * heads * NP,), jnp.float32)],
        scratch_types=[pltpu.VMEM((CE,), jnp.int32),
                       pltpu.VMEM((CE,), jnp.int32),
                       pltpu.VMEM((NP,), jnp.float32),
                       pltpu.VMEM((NP,), jnp.float32),
                       pltpu.VMEM((NP,), jnp.float32),
                       pltpu.VMEM((CE,), jnp.float32)],
        name=f"sc_edge_p_h{heads}")()


def _sc_reduce(heads):
    return functools.partial(
        pl.kernel, functools.partial(_reduce_body, heads),
        mesh=_mesh,
        compiler_params=pltpu.CompilerParams(needs_layout_passes=False),
        out_type=[jax.ShapeDtypeStruct((heads * NP,), jnp.float32)],
        scratch_types=[pltpu.VMEM((NW * RPW,), jnp.float32),
                       pltpu.VMEM((RPW,), jnp.float32)],
        name=f"sc_reduce_h{heads}")()


def _sc_norm(heads):
    return functools.partial(
        pl.kernel, functools.partial(_norm_body, heads),
        mesh=_mesh,
        compiler_params=pltpu.CompilerParams(needs_layout_passes=False),
        out_type=[jax.ShapeDtypeStruct((heads * EP_PAD,), jnp.float32)],
        scratch_types=[pltpu.VMEM((CE,), jnp.int32),
                       pltpu.VMEM((CE,), jnp.float32),
                       pltpu.VMEM((NP,), jnp.float32),
                       pltpu.VMEM((CE,), jnp.float32)],
        name=f"sc_norm_h{heads}")()


def _sc_agg(heads, d, nt, gb):
    return functools.partial(
        pl.kernel, functools.partial(_agg_body, heads, d, gb),
        mesh=_mesh,
        compiler_params=pltpu.CompilerParams(needs_layout_passes=False),
        out_type=[jax.ShapeDtypeStruct((heads * NP * d,), jnp.float32)],
        scratch_types=[pltpu.VMEM((CS,), jnp.int32),
                       pltpu.VMEM((CS,), jnp.int32),
                       pltpu.VMEM((2 * KMAX,), jnp.int32),
                       pltpu.VMEM((2 * KMAX,), jnp.int32),
                       pltpu.VMEM((2 * KMAX,), jnp.int32),
                       pltpu.VMEM((KMAX,), jnp.float32),
                       pltpu.VMEM((2 * gb, d), jnp.float32),
                       pltpu.VMEM((RH * d,), jnp.float32),
                       pltpu.SemaphoreType.DMA,
                       pltpu.SemaphoreType.DMA],
        name=f"sc_agg_h{heads}_d{d}")()


def kernel(x, edge_index, batch, gamma1, beta1, W1, att_src1, att_dst1, bias1,
           gamma2, beta2, W2, att_src2, att_dst2, bias2):
    f32 = jnp.float32
    sl = jnp.arange(N, dtype=edge_index.dtype)
    ei = jnp.concatenate([edge_index, jnp.stack([sl, sl])], axis=1)
    src = ei[0].astype(jnp.int32)
    dst = ei[1].astype(jnp.int32)
    pad = EP_PAD - EP
    src_p = jnp.concatenate([src, jnp.zeros((pad,), jnp.int32)])
    dst_p = jnp.concatenate([dst, jnp.full((pad,), DST_PAD, jnp.int32)])

    mean1, rstd1 = pl.pallas_call(
        _stats1_body,
        out_shape=[jax.ShapeDtypeStruct((1, F_IN), f32)] * 2,
        name="tc_stats1")(x)
    RB = 1000
    h_hm, asrc1_b, adst1_b = pl.pallas_call(
        _l1_body,
        grid=(N // RB,),
        in_specs=[pl.BlockSpec((RB, F_IN), lambda i: (i, 0)),
                  pl.BlockSpec((1, F_IN), lambda i: (0, 0)),
                  pl.BlockSpec((1, F_IN), lambda i: (0, 0)),
                  pl.BlockSpec((1, F_IN), lambda i: (0, 0)),
                  pl.BlockSpec((1, F_IN), lambda i: (0, 0)),
                  pl.BlockSpec((F_IN, HEADS * HID), lambda i: (0, 0)),
                  pl.BlockSpec((HEADS, HID), lambda i: (0, 0)),
                  pl.BlockSpec((HEADS, HID), lambda i: (0, 0))],
        out_specs=[pl.BlockSpec((HEADS, RB, HID), lambda i: (0, i, 0)),
                   pl.BlockSpec((1, HEADS, RB), lambda i: (i, 0, 0)),
                   pl.BlockSpec((1, HEADS, RB), lambda i: (i, 0, 0))],
        out_shape=[jax.ShapeDtypeStruct((HEADS, N, HID), f32),
                   jax.ShapeDtypeStruct((N // RB, HEADS, RB), f32),
                   jax.ShapeDtypeStruct((N // RB, HEADS, RB), f32)],
        name="tc_l1")(x, mean1, rstd1, gamma1.reshape(1, F_IN),
                      beta1.reshape(1, F_IN), W1, att_src1, att_dst1)
    asrc1 = asrc1_b.transpose(1, 0, 2).reshape(HEADS, N)
    adst1 = adst1_b.transpose(1, 0, 2).reshape(HEADS, N)

    p1, dparts1 = _sc_edge_p(HEADS)(src_p, dst_p, asrc1.reshape(-1),
                                    adst1.reshape(-1))
    dred1, = _sc_reduce(HEADS)(dparts1)
    alpha1_hm, = _sc_norm(HEADS)(dst_p, p1, dred1)
    out1_hm, = _sc_agg(HEADS, HID, N, 64)(src_p, dst_p, alpha1_hm, h_hm)
    out1_hm = out1_hm.reshape(HEADS, NP, HID)

    bias1_hm = bias1.reshape(HEADS, HID)
    mean2, rstd2 = pl.pallas_call(
        _stats2_body,
        grid=(HEADS,),
        in_specs=[pl.BlockSpec((1, NP, HID), lambda h: (h, 0, 0)),
                  pl.BlockSpec((1, 1, HID), lambda h: (h, 0, 0))],
        out_specs=[pl.BlockSpec((1, 1, HID), lambda h: (h, 0, 0)),
                   pl.BlockSpec((1, 1, HID), lambda h: (h, 0, 0))],
        out_shape=[jax.ShapeDtypeStruct((HEADS, 1, HID), f32)] * 2,
        name="tc_stats2")(out1_hm, bias1_hm.reshape(HEADS, 1, HID))
    mean2 = mean2.reshape(HEADS, HID)
    rstd2 = rstd2.reshape(HEADS, HID)

    W2_hm = jnp.pad(W2, ((0, 0), (0, DPAD - CLASSES))).reshape(HEADS, HID, DPAD)
    att_src2_p = jnp.pad(att_src2, ((0, 0), (0, DPAD - CLASSES)))
    att_dst2_p = jnp.pad(att_dst2, ((0, 0), (0, DPAD - CLASSES)))
    RB2 = 512
    h2_hm, asrc2_b, adst2_b = pl.pallas_call(
        _l2_body,
        grid=(NP // RB2,),
        in_specs=[pl.BlockSpec((HEADS, RB2, HID), lambda i: (0, i, 0)),
                  pl.BlockSpec((HEADS, HID), lambda i: (0, 0)),
                  pl.BlockSpec((HEADS, HID), lambda i: (0, 0)),
                  pl.BlockSpec((HEADS, HID), lambda i: (0, 0)),
                  pl.BlockSpec((HEADS, HID), lambda i: (0, 0)),
                  pl.BlockSpec((HEADS, HID), lambda i: (0, 0)),
                  pl.BlockSpec((HEADS, HID, DPAD), lambda i: (0, 0, 0)),
                  pl.BlockSpec((1, DPAD), lambda i: (0, 0)),
                  pl.BlockSpec((1, DPAD), lambda i: (0, 0))],
        out_specs=[pl.BlockSpec((1, RB2, DPAD), lambda i: (0, i, 0)),
                   pl.BlockSpec((1, 1, RB2), lambda i: (i, 0, 0)),
                   pl.BlockSpec((1, 1, RB2), lambda i: (i, 0, 0))],
        out_shape=[jax.ShapeDtypeStruct((1, NP, DPAD), f32),
                   jax.ShapeDtypeStruct((NP // RB2, 1, RB2), f32),
                   jax.ShapeDtypeStruct((NP // RB2, 1, RB2), f32)],
        name="tc_l2")(out1_hm, bias1_hm, mean2, rstd2,
                      gamma2.reshape(HEADS, HID), beta2.reshape(HEADS, HID),
                      W2_hm, att_src2_p, att_dst2_p)

    asrc2 = asrc2_b.reshape(1, NP)[:, :N]
    adst2 = adst2_b.reshape(1, NP)[:, :N]
    p2, dparts2 = _sc_edge_p(1)(src_p, dst_p, asrc2.reshape(-1),
                                adst2.reshape(-1))
    dred2, = _sc_reduce(1)(dparts2)
    alpha2_hm, = _sc_norm(1)(dst_p, p2, dred2)
    out2_hm, = _sc_agg(1, DPAD, NP, 128)(src_p, dst_p, alpha2_hm, h2_hm)
    out2_hm = out2_hm.reshape(1, NP, DPAD)

    batch_p = jnp.concatenate(
        [batch.astype(jnp.int32), jnp.full((NP - N,), NGRAPH, jnp.int32)]
    ).reshape(NP, 1)
    logp_pad = pl.pallas_call(
        _final_body,
        out_shape=jax.ShapeDtypeStruct((NGRAPH, DPAD), f32),
        name="tc_final")(out2_hm, jnp.pad(bias2, (0, DPAD - CLASSES))
                         .reshape(1, DPAD), batch_p)

    logp = logp_pad[:, :CLASSES]
    alpha1 = alpha1_hm.reshape(HEADS, EP_PAD).T[:EP]
    alpha2 = alpha2_hm.reshape(1, EP_PAD).T[:EP]
    return (logp, (ei, alpha1), (ei, alpha2))


# speedup vs baseline: 1.1510x; 1.1510x over previous
import functools

import jax
import jax.numpy as jnp
from jax import lax
from jax.experimental import pallas as pl
from jax.experimental.pallas import tpu as pltpu
from jax.experimental.pallas import tpu_sc as plsc

N = 10000
E = 160000
F_IN = 256
HID = 256
HEADS = 8
CLASSES = 10
NGRAPH = 16
EPS_BN = 1e-5
DPAD = 128

NC, NS, L = 2, 16, 16
NW = NC * NS
EP = E + N
CE = 5328
EP_PAD = CE * NW
NP = 10240
RPW = NP // NW
RH = RPW // 2
KMAX = 4096
CS = EP_PAD // 32
DST_PAD = N + 16

_mesh = plsc.VectorSubcoreMesh(core_axis_name="c", subcore_axis_name="s",
                               num_cores=NC, num_subcores=NS)


def _stats1_body(x_ref, mean_ref, rstd_ref):
    x = x_ref[...]
    mean = jnp.mean(x, axis=0, keepdims=True)
    var = jnp.mean((x - mean) ** 2, axis=0, keepdims=True)
    mean_ref[...] = mean
    rstd_ref[...] = lax.rsqrt(var + EPS_BN)


def _l1_body(x_ref, mean_ref, rstd_ref, g_ref, b_ref, w_ref, asr_ref, adr_ref,
             h_ref, as_ref, ad_ref):
    xb = (x_ref[...] - mean_ref[...]) * rstd_ref[...] * g_ref[...] + b_ref[...]
    for hd in range(HEADS):
        w = w_ref[:, hd * HID:(hd + 1) * HID]
        hh = jnp.dot(xb, w, preferred_element_type=jnp.float32)
        h_ref[hd] = hh
        as_ref[0, hd] = jnp.sum(hh * asr_ref[hd][None, :], axis=1)
        ad_ref[0, hd] = jnp.sum(hh * adr_ref[hd][None, :], axis=1)


def _stats2_body(h_ref, bias_ref, mean_ref, rstd_ref):
    y = jnp.maximum(h_ref[0] + bias_ref[0], 0.0)
    rmask = lax.broadcasted_iota(jnp.int32, (NP, 1), 0) < N
    y = jnp.where(rmask, y, 0.0)
    mean = jnp.sum(y, axis=0, keepdims=True) / N
    d = jnp.where(rmask, y - mean, 0.0)
    var = jnp.sum(d * d, axis=0, keepdims=True) / N
    mean_ref[0] = mean
    rstd_ref[0] = lax.rsqrt(var + EPS_BN)


def _l2_body(h_ref, bias_ref, mean_ref, rstd_ref, g_ref, b_ref, w_ref,
             asr_ref, adr_ref, h2_ref, as_ref, ad_ref):
    rb = h_ref.shape[1]
    acc = jnp.zeros((rb, DPAD), jnp.float32)
    for hd in range(HEADS):
        y = jnp.maximum(h_ref[hd] + bias_ref[hd][None, :], 0.0)
        yb = (y - mean_ref[hd][None, :]) * rstd_ref[hd][None, :] \
            * g_ref[hd][None, :] + b_ref[hd][None, :]
        acc = acc + jnp.dot(yb, w_ref[hd], preferred_element_type=jnp.float32)
    h2_ref[0] = acc
    as_ref[0, 0] = jnp.sum(acc * asr_ref[...], axis=1)
    ad_ref[0, 0] = jnp.sum(acc * adr_ref[...], axis=1)


def _final_body(o_ref, bias_ref, batch_ref, out_ref):
    act = o_ref[0] + bias_ref[...]
    act = jnp.where(act > 0.0, act, jnp.exp(act) - 1.0)
    gid = lax.broadcasted_iota(jnp.int32, (NP, NGRAPH), 1)
    oh = (batch_ref[...] == gid).astype(jnp.float32)
    sums = lax.dot_general(oh, act, (((0,), (0,)), ((), ())),
                           preferred_element_type=jnp.float32)
    counts = jnp.sum(oh, axis=0)
    pooled = sums / jnp.maximum(counts, 1.0)[:, None]
    cmask = lax.broadcasted_iota(jnp.int32, (NGRAPH, DPAD), 1) < CLASSES
    pm = jnp.where(cmask, pooled, -1e30)
    mx = jnp.max(pm, axis=1, keepdims=True)
    lse = jnp.log(jnp.sum(jnp.exp(pm - mx), axis=1, keepdims=True)) + mx
    out_ref[...] = pm - lse


def _edge_p_body(heads, src_hbm, dst_hbm, asrc_hbm, adst_hbm,
                 p_hbm, dparts_hbm,
                 src_v, dst_v, asrc_v, adst_v, den_v, p_v):
    wid = lax.axis_index("s") * NC + lax.axis_index("c")
    base = wid * CE
    pltpu.sync_copy(src_hbm.at[pl.ds(base, CE)], src_v)
    pltpu.sync_copy(dst_hbm.at[pl.ds(base, CE)], dst_v)
    zero16 = jnp.zeros((L,), jnp.float32)

    def head_body(hd, c):
        pltpu.sync_copy(asrc_hbm.at[pl.ds(hd * N, N)], asrc_v.at[pl.ds(0, N)])
        pltpu.sync_copy(adst_hbm.at[pl.ds(hd * N, N)], adst_v.at[pl.ds(0, N)])

        def ztail(i, c2):
            adst_v[pl.ds(N + i * L, L)] = zero16
            return c2
        lax.fori_loop(0, (NP - N) // L, ztail, 0)

        def zden(i, c2):
            den_v[pl.ds(i * L, L)] = zero16
            return c2
        lax.fori_loop(0, NP // L, zden, 0)

        def ebody(i, c2):
            sv = src_v[pl.ds(i * L, L)]
            dv = dst_v[pl.ds(i * L, L)]
            av = plsc.load_gather(asrc_v, [sv]) + plsc.load_gather(adst_v, [dv])
            z = jnp.where(av >= 0.0, av, av * 0.2)
            p = jnp.exp(z)
            p_v[pl.ds(i * L, L)] = p
            plsc.addupdate_scatter(den_v, [dv], p)
            return c2
        lax.fori_loop(0, CE // L, ebody, 0)
        pltpu.sync_copy(p_v, p_hbm.at[pl.ds(hd * EP_PAD + base, CE)])
        pltpu.sync_copy(den_v,
                        dparts_hbm.at[pl.ds((wid * heads + hd) * NP, NP)])
        return c
    lax.fori_loop(0, heads, head_body, 0)


def _reduce_body(heads, dparts_hbm, dred_hbm, parts_v, red_v):
    wid = lax.axis_index("s") * NC + lax.axis_index("c")
    lo = wid * RPW

    def head_body(hd, c):
        def ld(k, c2):
            pltpu.sync_copy(
                dparts_hbm.at[pl.ds((k * heads + hd) * NP + lo, RPW)],
                parts_v.at[pl.ds(k * RPW, RPW)])
            return c2
        lax.fori_loop(0, NW, ld, 0)

        def rbody(j, c2):
            acc = jnp.zeros((L,), jnp.float32)
            for k in range(NW):
                acc = acc + parts_v[pl.ds(k * RPW + j * L, L)]
            red_v[pl.ds(j * L, L)] = acc
            return c2
        lax.fori_loop(0, RPW // L, rbody, 0)
        pltpu.sync_copy(red_v, dred_hbm.at[pl.ds(hd * NP + lo, RPW)])
        return c
    lax.fori_loop(0, heads, head_body, 0)


def _norm_body(heads, dst_hbm, p_hbm, dred_hbm, alpha_hbm,
               dst_v, p_v, den_v, al_v):
    wid = lax.axis_index("s") * NC + lax.axis_index("c")
    base = wid * CE
    pltpu.sync_copy(dst_hbm.at[pl.ds(base, CE)], dst_v)

    def head_body(hd, c):
        pltpu.sync_copy(dred_hbm.at[pl.ds(hd * NP, NP)], den_v)
        pltpu.sync_copy(p_hbm.at[pl.ds(hd * EP_PAD + base, CE)], p_v)

        def ebody(i, c2):
            dv = dst_v[pl.ds(i * L, L)]
            p = p_v[pl.ds(i * L, L)]
            dsum = plsc.load_gather(den_v, [dv])
            al_v[pl.ds(i * L, L)] = p / (dsum + 1e-16)
            return c2
        lax.fori_loop(0, CE // L, ebody, 0)
        pltpu.sync_copy(al_v, alpha_hbm.at[pl.ds(hd * EP_PAD + base, CE)])
        return c
    lax.fori_loop(0, heads, head_body, 0)


def _agg_body(heads, d, gb, src_hbm, dst_hbm, alpha_hbm, h_hbm, out_hbm,
              ssrc_v, sdst_v, ksrc_v, kloc_v, keid_v, abuf_v, gbuf_v, acc_v,
              sem0, sem1):
    wid = lax.axis_index("s") * NC + lax.axis_index("c")
    lo = wid * RPW
    lane = lax.iota(jnp.int32, L)
    zero16i = jnp.zeros((L,), jnp.int32)
    zero16f = jnp.zeros((L,), jnp.float32)

    def zk(i, c):
        ksrc_v[pl.ds(i * L, L)] = zero16i
        kloc_v[pl.ds(i * L, L)] = zero16i
        keid_v[pl.ds(i * L, L)] = zero16i
        return c
    lax.fori_loop(0, 2 * KMAX // L, zk, 0)

    def scan_chunk(s, cnts):
        pltpu.sync_copy(src_hbm.at[pl.ds(s * CS, CS)], ssrc_v)
        pltpu.sync_copy(dst_hbm.at[pl.ds(s * CS, CS)], sdst_v)

        def sbody(j, cnts2):
            c0, c1 = cnts2
            sv = ssrc_v[pl.ds(j * L, L)]
            dv = sdst_v[pl.ds(j * L, L)]
            eid = lax.iota(jnp.int32, L) + (s * CS + j * L)
            rel = dv - lo
            m0 = (rel >= 0) & (rel < RH)
            m1 = (rel >= RH) & (rel < RPW)
            plsc.store_compressed(ksrc_v.at[pl.ds(c0, L)], sv, mask=m0)
            plsc.store_compressed(kloc_v.at[pl.ds(c0, L)], rel, mask=m0)
            plsc.store_compressed(keid_v.at[pl.ds(c0, L)], eid, mask=m0)
            plsc.store_compressed(ksrc_v.at[pl.ds(KMAX + c1, L)], sv, mask=m1)
            plsc.store_compressed(kloc_v.at[pl.ds(KMAX + c1, L)], rel - RH,
                                  mask=m1)
            plsc.store_compressed(keid_v.at[pl.ds(KMAX + c1, L)], eid, mask=m1)
            c0 = jnp.minimum(c0 + jnp.sum(m0.astype(jnp.int32)), KMAX - L)
            c1 = jnp.minimum(c1 + jnp.sum(m1.astype(jnp.int32)), KMAX - L)
            return (c0, c1)
        return lax.fori_loop(0, CS // L, sbody, cnts)

    cnt0, cnt1 = lax.fori_loop(0, EP_PAD // CS, scan_chunk,
                               (jnp.int32(0), jnp.int32(0)))

    def run_half(hd, hf, cnt):
        def zacc(i, c):
            acc_v[pl.ds(i * L, L)] = zero16f
            return c
        lax.fori_loop(0, RH * d // L, zacc, 0)

        atab = alpha_hbm.at[pl.ds(hd * EP_PAD, EP_PAD)]
        pltpu.async_copy(atab.at[keid_v.at[pl.ds(hf * KMAX, KMAX)]],
                         abuf_v, sem0).wait()

        htab = h_hbm.at[hd]
        nch = (cnt + gb - 1) // gb

        @pl.when(nch > 0)
        def _():
            idx = ksrc_v.at[pl.ds(hf * KMAX, gb)]
            pltpu.async_copy(htab.at[idx], gbuf_v.at[pl.ds(0, gb)], sem0)

        def chunk(jc, c):
            even = (jc % 2) == 0
            nxt = jc + 1

            @pl.when((nxt < nch) & even)
            def _():
                idx = ksrc_v.at[pl.ds(hf * KMAX + nxt * gb, gb)]
                pltpu.async_copy(htab.at[idx], gbuf_v.at[pl.ds(gb, gb)], sem1)

            @pl.when((nxt < nch) & (~even))
            def _():
                idx = ksrc_v.at[pl.ds(hf * KMAX + nxt * gb, gb)]
                pltpu.async_copy(htab.at[idx], gbuf_v.at[pl.ds(0, gb)], sem0)

            @pl.when(even)
            def _():
                pltpu.make_async_copy(htab.at[pl.ds(0, gb)],
                                      gbuf_v.at[pl.ds(0, gb)], sem0).wait()

            @pl.when(~even)
            def _():
                pltpu.make_async_copy(htab.at[pl.ds(0, gb)],
                                      gbuf_v.at[pl.ds(gb, gb)], sem1).wait()

            rowbase = (jc % 2) * gb

            def group(q, c2):
                ebase = jc * gb + q * L
                lv = kloc_v[pl.ds(hf * KMAX + ebase, L)]
                av = abuf_v[pl.ds(ebase, L)]
                av = jnp.where(ebase + lane < cnt, av, 0.0)
                offv = lv * d
                for r in range(L):
                    msk = lane == r
                    off = jnp.sum(jnp.where(msk, offv, 0))
                    ab = jnp.full((L,), jnp.sum(jnp.where(msk, av, 0.0)))
                    row = rowbase + q * L + r
                    for g in range(d // L):
                        x = gbuf_v[row, pl.ds(g * L, L)].reshape((L,)) * ab
                        plsc.addupdate(acc_v.at[pl.ds(off + g * L, L)], x)
                return c2
            lax.fori_loop(0, gb // L, group, 0)
            return c
        lax.fori_loop(0, nch, chunk, 0)
        pltpu.sync_copy(
            acc_v,
            out_hbm.at[pl.ds((hd * NP + lo + hf * RH) * d, RH * d)])

    def head_body(hd, c):
        run_half(hd, 0, cnt0)
        run_half(hd, 1, cnt1)
        return c
    lax.fori_loop(0, heads, head_body, 0)


def _sc_edge_p(heads):
    return functools.partial(
        pl.kernel, functools.partial(_edge_p_body, heads),
        mesh=_mesh,
        compiler_params=pltpu.CompilerParams(needs_layout_passes=False),
        out_type=[jax.ShapeDtypeStruct((heads * EP_PAD,), jnp.float32),
                  jax.ShapeDtypeStruct((NW * heads * NP,), jnp.float32)],
        scratch_types=[pltpu.VMEM((CE,), jnp.int32),
                       pltpu.VMEM((CE,), jnp.int32),
                       pltpu.VMEM((NP,), jnp.float32),
                       pltpu.VMEM((NP,), jnp.float32),
                       pltpu.VMEM((NP,), jnp.float32),
                       pltpu.VMEM((CE,), jnp.float32)],
        name=f"sc_edge_p_h{heads}")()


def _sc_reduce(heads):
    return functools.partial(
        pl.kernel, functools.partial(_reduce_body, heads),
        mesh=_mesh,
        compiler_params=pltpu.CompilerParams(needs_layout_passes=False),
        out_type=[jax.ShapeDtypeStruct((heads * NP,), jnp.float32)],
        scratch_types=[pltpu.VMEM((NW * RPW,), jnp.float32),
                       pltpu.VMEM((RPW,), jnp.float32)],
        name=f"sc_reduce_h{heads}")()


def _sc_norm(heads):
    return functools.partial(
        pl.kernel, functools.partial(_norm_body, heads),
        mesh=_mesh,
        compiler_params=pltpu.CompilerParams(needs_layout_passes=False),
        out_type=[jax.ShapeDtypeStruct((heads * EP_PAD,), jnp.float32)],
        scratch_types=[pltpu.VMEM((CE,), jnp.int32),
                       pltpu.VMEM((CE,), jnp.float32),
                       pltpu.VMEM((NP,), jnp.float32),
                       pltpu.VMEM((CE,), jnp.float32)],
        name=f"sc_norm_h{heads}")()


def _sc_agg(heads, d, nt, gb):
    return functools.partial(
        pl.kernel, functools.partial(_agg_body, heads, d, gb),
        mesh=_mesh,
        compiler_params=pltpu.CompilerParams(needs_layout_passes=False),
        out_type=[jax.ShapeDtypeStruct((heads * NP * d,), jnp.float32)],
        scratch_types=[pltpu.VMEM((CS,), jnp.int32),
                       pltpu.VMEM((CS,), jnp.int32),
                       pltpu.VMEM((2 * KMAX,), jnp.int32),
                       pltpu.VMEM((2 * KMAX,), jnp.int32),
                       pltpu.VMEM((2 * KMAX,), jnp.int32),
                       pltpu.VMEM((KMAX,), jnp.float32),
                       pltpu.VMEM((2 * gb, d), jnp.float32),
                       pltpu.VMEM((RH * d,), jnp.float32),
                       pltpu.SemaphoreType.DMA,
                       pltpu.SemaphoreType.DMA],
        name=f"sc_agg_h{heads}_d{d}")()


def kernel(x, edge_index, batch, gamma1, beta1, W1, att_src1, att_dst1, bias1,
           gamma2, beta2, W2, att_src2, att_dst2, bias2):
    f32 = jnp.float32
    sl = jnp.arange(N, dtype=edge_index.dtype)
    ei = jnp.concatenate([edge_index, jnp.stack([sl, sl])], axis=1)
    src = ei[0].astype(jnp.int32)
    dst = ei[1].astype(jnp.int32)
    pad = EP_PAD - EP
    src_p = jnp.concatenate([src, jnp.zeros((pad,), jnp.int32)])
    dst_p = jnp.concatenate([dst, jnp.full((pad,), DST_PAD, jnp.int32)])

    mean1, rstd1 = pl.pallas_call(
        _stats1_body,
        out_shape=[jax.ShapeDtypeStruct((1, F_IN), f32)] * 2,
        name="tc_stats1")(x)
    RB = 1000
    h_hm, asrc1_b, adst1_b = pl.pallas_call(
        _l1_body,
        grid=(N // RB,),
        in_specs=[pl.BlockSpec((RB, F_IN), lambda i: (i, 0)),
                  pl.BlockSpec((1, F_IN), lambda i: (0, 0)),
                  pl.BlockSpec((1, F_IN), lambda i: (0, 0)),
                  pl.BlockSpec((1, F_IN), lambda i: (0, 0)),
                  pl.BlockSpec((1, F_IN), lambda i: (0, 0)),
                  pl.BlockSpec((F_IN, HEADS * HID), lambda i: (0, 0)),
                  pl.BlockSpec((HEADS, HID), lambda i: (0, 0)),
                  pl.BlockSpec((HEADS, HID), lambda i: (0, 0))],
        out_specs=[pl.BlockSpec((HEADS, RB, HID), lambda i: (0, i, 0)),
                   pl.BlockSpec((1, HEADS, RB), lambda i: (i, 0, 0)),
                   pl.BlockSpec((1, HEADS, RB), lambda i: (i, 0, 0))],
        out_shape=[jax.ShapeDtypeStruct((HEADS, N, HID), f32),
                   jax.ShapeDtypeStruct((N // RB, HEADS, RB), f32),
                   jax.ShapeDtypeStruct((N // RB, HEADS, RB), f32)],
        name="tc_l1")(x, mean1, rstd1, gamma1.reshape(1, F_IN),
                      beta1.reshape(1, F_IN), W1, att_src1, att_dst1)
    asrc1 = asrc1_b.transpose(1, 0, 2).reshape(HEADS, N)
    adst1 = adst1_b.transpose(1, 0, 2).reshape(HEADS, N)

    p1, dparts1 = _sc_edge_p(HEADS)(src_p, dst_p, asrc1.reshape(-1),
                                    adst1.reshape(-1))
    dred1, = _sc_reduce(HEADS)(dparts1)
    alpha1_hm, = _sc_norm(HEADS)(dst_p, p1, dred1)
    out1_hm, = _sc_agg(HEADS, HID, N, 64)(src_p, dst_p, alpha1_hm, h_hm)
    out1_hm = out1_hm.reshape(HEADS, NP, HID)

    bias1_hm = bias1.reshape(HEADS, HID)
    mean2, rstd2 = pl.pallas_call(
        _stats2_body,
        grid=(HEADS,),
        in_specs=[pl.BlockSpec((1, NP, HID), lambda h: (h, 0, 0)),
                  pl.BlockSpec((1, 1, HID), lambda h: (h, 0, 0))],
        out_specs=[pl.BlockSpec((1, 1, HID), lambda h: (h, 0, 0)),
                   pl.BlockSpec((1, 1, HID), lambda h: (h, 0, 0))],
        out_shape=[jax.ShapeDtypeStruct((HEADS, 1, HID), f32)] * 2,
        name="tc_stats2")(out1_hm, bias1_hm.reshape(HEADS, 1, HID))
    mean2 = mean2.reshape(HEADS, HID)
    rstd2 = rstd2.reshape(HEADS, HID)

    W2_hm = jnp.pad(W2, ((0, 0), (0, DPAD - CLASSES))).reshape(HEADS, HID, DPAD)
    att_src2_p = jnp.pad(att_src2, ((0, 0), (0, DPAD - CLASSES)))
    att_dst2_p = jnp.pad(att_dst2, ((0, 0), (0, DPAD - CLASSES)))
    RB2 = 512
    h2_hm, asrc2_b, adst2_b = pl.pallas_call(
        _l2_body,
        grid=(NP // RB2,),
        in_specs=[pl.BlockSpec((HEADS, RB2, HID), lambda i: (0, i, 0)),
                  pl.BlockSpec((HEADS, HID), lambda i: (0, 0)),
                  pl.BlockSpec((HEADS, HID), lambda i: (0, 0)),
                  pl.BlockSpec((HEADS, HID), lambda i: (0, 0)),
                  pl.BlockSpec((HEADS, HID), lambda i: (0, 0)),
                  pl.BlockSpec((HEADS, HID), lambda i: (0, 0)),
                  pl.BlockSpec((HEADS, HID, DPAD), lambda i: (0, 0, 0)),
                  pl.BlockSpec((1, DPAD), lambda i: (0, 0)),
                  pl.BlockSpec((1, DPAD), lambda i: (0, 0))],
        out_specs=[pl.BlockSpec((1, RB2, DPAD), lambda i: (0, i, 0)),
                   pl.BlockSpec((1, 1, RB2), lambda i: (i, 0, 0)),
                   pl.BlockSpec((1, 1, RB2), lambda i: (i, 0, 0))],
        out_shape=[jax.ShapeDtypeStruct((1, NP, DPAD), f32),
                   jax.ShapeDtypeStruct((NP // RB2, 1, RB2), f32),
                   jax.ShapeDtypeStruct((NP // RB2, 1, RB2), f32)],
        name="tc_l2")(out1_hm, bias1_hm, mean2, rstd2,
                      gamma2.reshape(HEADS, HID), beta2.reshape(HEADS, HID),
                      W2_hm, att_src2_p, att_dst2_p)

    asrc2 = asrc2_b.reshape(1, NP)[:, :N]
    adst2 = adst2_b.reshape(1, NP)[:, :N]
    p2, dparts2 = _sc_edge_p(1)(src_p, dst_p, asrc2.reshape(-1),
                                adst2.reshape(-1))
    dred2, = _sc_reduce(1)(dparts2)
    alpha2_hm, = _sc_norm(1)(dst_p, p2, dred2)
    out2_hm, = _sc_agg(1, DPAD, NP, 128)(src_p, dst_p, alpha2_hm, h2_hm)
    out2_hm = out2_hm.reshape(1, NP, DPAD)

    batch_p = jnp.concatenate(
        [batch.astype(jnp.int32), jnp.full((NP - N,), NGRAPH, jnp.int32)]
    ).reshape(NP, 1)
    logp_pad = pl.pallas_call(
        _final_body,
        out_shape=jax.ShapeDtypeStruct((NGRAPH, DPAD), f32),
        name="tc_final")(out2_hm, jnp.pad(bias2, (0, DPAD - CLASSES))
                         .reshape(1, DPAD), batch_p)

    logp = logp_pad[:, :CLASSES]
    alpha1 = alpha1_hm.reshape(HEADS, EP_PAD).T[:EP]
    alpha2 = alpha2_hm.reshape(1, EP_PAD).T[:EP]
    return (logp, (ei, alpha1), (ei, alpha2))
